# Initial kernel scaffold; baseline (speedup 1.0000x reference)
import jax
import jax.numpy as jnp
from jax.experimental import pallas as pl


def kernel(x, edge_index, W1, b1, W2, b2, fc_W, fc_b):
    raise NotImplementedError("write your pallas kernel here")


# speedup vs baseline: 29.5182x; 29.5182x over previous
import functools

import jax
import jax.numpy as jnp
from jax import lax
from jax.experimental import pallas as pl
from jax.experimental.pallas import tpu as pltpu
from jax.experimental.pallas import tpu_sc as plsc

N_NODES = 10000
DEG_W = 16
N_PAD = 10240
NC, NS = 2, 16
NW = NC * NS
EB = 128
RPT = N_PAD // NS


def _sc_mesh():
    return plsc.VectorSubcoreMesh(
        core_axis_name="c", subcore_axis_name="s",
        num_cores=NC, num_subcores=NS)


def _num_batches(w, n_edges, nb_max):
    per_tile = nb_max * EB
    real = jnp.clip(n_edges - w * per_tile, 0, per_tile)
    return (real + EB - 1) // EB


def _make_deg(n_edges, nb_max):
    @functools.partial(
        pl.kernel,
        out_type=jax.ShapeDtypeStruct((NC, N_PAD, DEG_W), jnp.float32),
        mesh=_sc_mesh(),
        compiler_params=pltpu.CompilerParams(use_tc_tiling_on_sc=False),
        scratch_types=[
            pltpu.VMEM((nb_max, EB), jnp.int32),
            pltpu.VMEM((EB, DEG_W), jnp.float32),
            pltpu.VMEM_SHARED((N_PAD, DEG_W), jnp.float32),
        ],
    )
    def deg_k(dst_hbm, ones_hbm, zeros_hbm, out_hbm, dst_v, ones_v, acc_sh):
        cid = lax.axis_index("c")
        sid = lax.axis_index("s")
        w = cid * NS + sid
        pltpu.sync_copy(dst_hbm.at[w], dst_v)
        pltpu.sync_copy(ones_hbm, ones_v)
        pltpu.sync_copy(zeros_hbm, acc_sh.at[pl.ds(sid * RPT, RPT)])
        plsc.subcore_barrier()
        nb = _num_batches(w, n_edges, nb_max)

        def body(j, carry):
            pltpu.sync_copy(ones_v, acc_sh.at[dst_v.at[j]], add=True)
            return carry

        lax.fori_loop(0, nb, body, 0)
        plsc.subcore_barrier()
        pltpu.sync_copy(acc_sh.at[pl.ds(sid * RPT, RPT)],
                        out_hbm.at[cid, pl.ds(sid * RPT, RPT)])

    return deg_k


def _make_agg(d_feat, n_edges, nb_max):
    @functools.partial(
        pl.kernel,
        out_type=jax.ShapeDtypeStruct((NC, N_PAD, d_feat), jnp.float32),
        mesh=_sc_mesh(),
        compiler_params=pltpu.CompilerParams(use_tc_tiling_on_sc=False),
        scratch_types=[
            pltpu.VMEM((nb_max, EB), jnp.int32),
            pltpu.VMEM((nb_max, EB), jnp.int32),
            pltpu.VMEM((EB, d_feat), jnp.float32),
            pltpu.VMEM_SHARED((N_PAD, d_feat), jnp.float32),
            pltpu.SemaphoreType.DMA,
        ],
    )
    def agg_k(g_hbm, src_hbm, dst_hbm, zeros_hbm, out_hbm,
              src_v, dst_v, rows_v, acc_sh, sem):
        cid = lax.axis_index("c")
        sid = lax.axis_index("s")
        w = cid * NS + sid
        pltpu.sync_copy(src_hbm.at[w], src_v)
        pltpu.sync_copy(dst_hbm.at[w], dst_v)
        pltpu.sync_copy(zeros_hbm, acc_sh.at[pl.ds(sid * RPT, RPT)])
        plsc.subcore_barrier()
        nb = _num_batches(w, n_edges, nb_max)

        def body(j, carry):
            pltpu.async_copy(g_hbm.at[src_v.at[j]], rows_v, sem).wait()
            pltpu.sync_copy(rows_v, acc_sh.at[dst_v.at[j]], add=True)
            return carry

        lax.fori_loop(0, nb, body, 0)
        plsc.subcore_barrier()
        pltpu.sync_copy(acc_sh.at[pl.ds(sid * RPT, RPT)],
                        out_hbm.at[cid, pl.ds(sid * RPT, RPT)])

    return agg_k


def _tc_prep1(deg2, x, w1):
    n, d_in = x.shape
    d_out = w1.shape[1]

    def body(deg_ref, x_ref, w_ref, dinv_ref, h_ref, g_ref):
        nr = x_ref.shape[0]
        deg = deg_ref[0, :nr, 0:1] + deg_ref[1, :nr, 0:1] + 1.0
        dinv = lax.rsqrt(deg)
        h = jnp.dot(x_ref[...], w_ref[...], preferred_element_type=jnp.float32,
                    precision=lax.Precision.HIGHEST)
        dinv_ref[...] = dinv
        h_ref[...] = h
        g_ref[...] = h * dinv

    return pl.pallas_call(
        body,
        out_shape=(
            jax.ShapeDtypeStruct((n, 1), jnp.float32),
            jax.ShapeDtypeStruct((n, d_out), jnp.float32),
            jax.ShapeDtypeStruct((n, d_out), jnp.float32),
        ),
    )(deg2, x, w1)


def _tc_mid(agg2, h1, dinv, b1, w2):
    n, d1 = h1.shape
    d2 = w2.shape[1]

    def body(agg_ref, h1_ref, dinv_ref, b1_ref, w2_ref, h2_ref, g2_ref):
        dinv_v = dinv_ref[...]
        n_rows = h1_ref.shape[0]
        pre = (dinv_v * (agg_ref[0, :n_rows] + agg_ref[1, :n_rows])
               + (dinv_v * dinv_v) * h1_ref[...] + b1_ref[...])
        out1 = jnp.maximum(pre, 0.0)
        h2 = jnp.dot(out1, w2_ref[...], preferred_element_type=jnp.float32,
                    precision=lax.Precision.HIGHEST)
        h2_ref[...] = h2
        g2_ref[...] = h2 * dinv_v

    return pl.pallas_call(
        body,
        out_shape=(
            jax.ShapeDtypeStruct((n, d2), jnp.float32),
            jax.ShapeDtypeStruct((n, d2), jnp.float32),
        ),
    )(agg2, h1, dinv, b1, w2)


def _tc_final(agg2, h2, dinv, b2, fc_w, fc_b):
    n, d2 = h2.shape

    def body(agg_ref, h2_ref, dinv_ref, b2_ref, fcw_ref, fcb_ref, out_ref):
        dinv_v = dinv_ref[...]
        n_rows = h2_ref.shape[0]
        pre = (dinv_v * (agg_ref[0, :n_rows] + agg_ref[1, :n_rows])
               + (dinv_v * dinv_v) * h2_ref[...] + b2_ref[...])
        out2 = jnp.maximum(pre, 0.0)
        pooled = jnp.sum(out2, axis=0, keepdims=True) * (1.0 / n)
        val = jnp.dot(pooled, fcw_ref[...], preferred_element_type=jnp.float32,
                    precision=lax.Precision.HIGHEST)
        out_ref[...] = val + fcb_ref[...]

    return pl.pallas_call(
        body,
        out_shape=jax.ShapeDtypeStruct((1, 1), jnp.float32),
    )(agg2, h2, dinv, b2, fc_w, fc_b)


def kernel(x, edge_index, W1, b1, W2, b2, fc_W, fc_b):
    n_edges = edge_index.shape[1]
    chunk = NW * EB
    e_pad = ((n_edges + chunk - 1) // chunk) * chunk
    nb_max = e_pad // chunk
    pad = e_pad - n_edges
    src = jnp.concatenate(
        [edge_index[0], jnp.zeros((pad,), jnp.int32)]).reshape(NW, nb_max, EB)
    dst = jnp.concatenate(
        [edge_index[1], jnp.zeros((pad,), jnp.int32)]).reshape(NW, nb_max, EB)

    ones_col = jnp.ones((EB, DEG_W), jnp.float32)
    zeros_col = jnp.zeros((RPT, DEG_W), jnp.float32)

    deg2 = _make_deg(n_edges, nb_max)(dst, ones_col, zeros_col)
    dinv, h1, g1 = _tc_prep1(deg2, x, W1)

    d1 = W1.shape[1]
    zeros1 = jnp.zeros((RPT, d1), jnp.float32)
    agg1 = _make_agg(d1, n_edges, nb_max)(g1, src, dst, zeros1)
    h2, g2 = _tc_mid(agg1, h1, dinv, b1, W2)

    d2 = W2.shape[1]
    zeros2 = jnp.zeros((RPT, d2), jnp.float32)
    agg2 = _make_agg(d2, n_edges, nb_max)(g2, src, dst, zeros2)
    out = _tc_final(agg2, h2, dinv, b2, fc_W, fc_b)
    return out.reshape(1)


# speedup vs baseline: 33.9176x; 1.1490x over previous
import functools

import jax
import jax.numpy as jnp
from jax import lax
from jax.experimental import pallas as pl
from jax.experimental.pallas import tpu as pltpu
from jax.experimental.pallas import tpu_sc as plsc

N_NODES = 10000
DEG_W = 16
N_PAD = 10240
NC, NS = 2, 16
NW = NC * NS
EB = 128
RPT = N_PAD // NS


def _sc_mesh():
    return plsc.VectorSubcoreMesh(
        core_axis_name="c", subcore_axis_name="s",
        num_cores=NC, num_subcores=NS)


def _num_batches(w, n_edges, nb_max):
    per_tile = nb_max * EB
    real = jnp.clip(n_edges - w * per_tile, 0, per_tile)
    return (real + EB - 1) // EB


def _make_deg(n_edges, nb_max):
    @functools.partial(
        pl.kernel,
        out_type=jax.ShapeDtypeStruct((NC, N_PAD, DEG_W), jnp.float32),
        mesh=_sc_mesh(),
        compiler_params=pltpu.CompilerParams(use_tc_tiling_on_sc=False),
        scratch_types=[
            pltpu.VMEM((nb_max, EB), jnp.int32),
            pltpu.VMEM((EB, DEG_W), jnp.float32),
            pltpu.VMEM_SHARED((N_PAD, DEG_W), jnp.float32),
        ],
    )
    def deg_k(dst_hbm, ones_hbm, zeros_hbm, out_hbm, dst_v, ones_v, acc_sh):
        cid = lax.axis_index("c")
        sid = lax.axis_index("s")
        w = cid * NS + sid
        pltpu.sync_copy(dst_hbm.at[w], dst_v)
        pltpu.sync_copy(ones_hbm, ones_v)
        pltpu.sync_copy(zeros_hbm, acc_sh.at[pl.ds(sid * RPT, RPT)])
        plsc.subcore_barrier()
        nb = _num_batches(w, n_edges, nb_max)

        def body(j, carry):
            pltpu.sync_copy(ones_v, acc_sh.at[dst_v.at[j]], add=True)
            return carry

        lax.fori_loop(0, nb, body, 0)
        plsc.subcore_barrier()
        pltpu.sync_copy(acc_sh.at[pl.ds(sid * RPT, RPT)],
                        out_hbm.at[cid, pl.ds(sid * RPT, RPT)])

    return deg_k


def _make_agg(d_feat, n_edges, nb_max):
    @functools.partial(
        pl.kernel,
        out_type=jax.ShapeDtypeStruct((NC, N_PAD, d_feat), jnp.float32),
        mesh=_sc_mesh(),
        compiler_params=pltpu.CompilerParams(use_tc_tiling_on_sc=False),
        scratch_types=[
            pltpu.VMEM((nb_max, EB), jnp.int32),
            pltpu.VMEM((nb_max, EB), jnp.int32),
            pltpu.VMEM((EB, d_feat), jnp.float32),
            pltpu.VMEM((EB, d_feat), jnp.float32),
            pltpu.VMEM_SHARED((N_PAD, d_feat), jnp.float32),
            pltpu.SemaphoreType.DMA,
            pltpu.SemaphoreType.DMA,
        ],
    )
    def agg_k(g_hbm, src_hbm, dst_hbm, zeros_hbm, out_hbm,
              src_v, dst_v, rows0, rows1, acc_sh, sem0, sem1):
        cid = lax.axis_index("c")
        sid = lax.axis_index("s")
        w = cid * NS + sid
        pltpu.sync_copy(src_hbm.at[w], src_v)
        pltpu.sync_copy(dst_hbm.at[w], dst_v)
        pltpu.sync_copy(zeros_hbm, acc_sh.at[pl.ds(sid * RPT, RPT)])
        plsc.subcore_barrier()
        nb = _num_batches(w, n_edges, nb_max)

        @pl.when(nb > 0)
        def _():
            pltpu.async_copy(g_hbm.at[src_v.at[0]], rows0, sem0)

        def stage(j, rows_cur, sem_cur, rows_nxt, sem_nxt):
            pltpu.make_async_copy(g_hbm.at[src_v.at[j]], rows_cur,
                                  sem_cur).wait()

            @pl.when(j + 1 < nb)
            def _():
                pltpu.async_copy(g_hbm.at[src_v.at[j + 1]], rows_nxt, sem_nxt)

            pltpu.sync_copy(rows_cur, acc_sh.at[dst_v.at[j]], add=True)

        def body(j, carry):
            @pl.when(j % 2 == 0)
            def _():
                stage(j, rows0, sem0, rows1, sem1)

            @pl.when(j % 2 == 1)
            def _():
                stage(j, rows1, sem1, rows0, sem0)

            return carry

        lax.fori_loop(0, nb, body, 0)
        plsc.subcore_barrier()
        pltpu.sync_copy(acc_sh.at[pl.ds(sid * RPT, RPT)],
                        out_hbm.at[cid, pl.ds(sid * RPT, RPT)])

    return agg_k


def _tc_prep1(deg2, x, w1):
    n, d_in = x.shape
    d_out = w1.shape[1]

    def body(deg_ref, x_ref, w_ref, dinv_ref, h_ref, g_ref):
        nr = x_ref.shape[0]
        deg = deg_ref[0, :nr, 0:1] + deg_ref[1, :nr, 0:1] + 1.0
        dinv = lax.rsqrt(deg)
        h = jnp.dot(x_ref[...], w_ref[...], preferred_element_type=jnp.float32,
                    precision=lax.Precision.HIGHEST)
        dinv_ref[...] = dinv
        h_ref[...] = h
        g_ref[...] = h * dinv

    return pl.pallas_call(
        body,
        out_shape=(
            jax.ShapeDtypeStruct((n, 1), jnp.float32),
            jax.ShapeDtypeStruct((n, d_out), jnp.float32),
            jax.ShapeDtypeStruct((n, d_out), jnp.float32),
        ),
    )(deg2, x, w1)


def _tc_mid(agg2, h1, dinv, b1, w2):
    n, d1 = h1.shape
    d2 = w2.shape[1]

    def body(agg_ref, h1_ref, dinv_ref, b1_ref, w2_ref, h2_ref, g2_ref):
        dinv_v = dinv_ref[...]
        n_rows = h1_ref.shape[0]
        pre = (dinv_v * (agg_ref[0, :n_rows] + agg_ref[1, :n_rows])
               + (dinv_v * dinv_v) * h1_ref[...] + b1_ref[...])
        out1 = jnp.maximum(pre, 0.0)
        h2 = jnp.dot(out1, w2_ref[...], preferred_element_type=jnp.float32,
                    precision=lax.Precision.HIGHEST)
        h2_ref[...] = h2
        g2_ref[...] = h2 * dinv_v

    return pl.pallas_call(
        body,
        out_shape=(
            jax.ShapeDtypeStruct((n, d2), jnp.float32),
            jax.ShapeDtypeStruct((n, d2), jnp.float32),
        ),
    )(agg2, h1, dinv, b1, w2)


def _tc_final(agg2, h2, dinv, b2, fc_w, fc_b):
    n, d2 = h2.shape

    def body(agg_ref, h2_ref, dinv_ref, b2_ref, fcw_ref, fcb_ref, out_ref):
        dinv_v = dinv_ref[...]
        n_rows = h2_ref.shape[0]
        pre = (dinv_v * (agg_ref[0, :n_rows] + agg_ref[1, :n_rows])
               + (dinv_v * dinv_v) * h2_ref[...] + b2_ref[...])
        out2 = jnp.maximum(pre, 0.0)
        pooled = jnp.sum(out2, axis=0, keepdims=True) * (1.0 / n)
        val = jnp.dot(pooled, fcw_ref[...], preferred_element_type=jnp.float32,
                    precision=lax.Precision.HIGHEST)
        out_ref[...] = val + fcb_ref[...]

    return pl.pallas_call(
        body,
        out_shape=jax.ShapeDtypeStruct((1, 1), jnp.float32),
    )(agg2, h2, dinv, b2, fc_w, fc_b)


def kernel(x, edge_index, W1, b1, W2, b2, fc_W, fc_b):
    n_edges = edge_index.shape[1]
    chunk = NW * EB
    e_pad = ((n_edges + chunk - 1) // chunk) * chunk
    nb_max = e_pad // chunk
    pad = e_pad - n_edges
    src = jnp.concatenate(
        [edge_index[0], jnp.zeros((pad,), jnp.int32)]).reshape(NW, nb_max, EB)
    dst = jnp.concatenate(
        [edge_index[1], jnp.zeros((pad,), jnp.int32)]).reshape(NW, nb_max, EB)

    ones_col = jnp.ones((EB, DEG_W), jnp.float32)
    zeros_col = jnp.zeros((RPT, DEG_W), jnp.float32)

    deg2 = _make_deg(n_edges, nb_max)(dst, ones_col, zeros_col)
    dinv, h1, g1 = _tc_prep1(deg2, x, W1)

    d1 = W1.shape[1]
    zeros1 = jnp.zeros((RPT, d1), jnp.float32)
    agg1 = _make_agg(d1, n_edges, nb_max)(g1, src, dst, zeros1)
    h2, g2 = _tc_mid(agg1, h1, dinv, b1, W2)

    d2 = W2.shape[1]
    zeros2 = jnp.zeros((RPT, d2), jnp.float32)
    agg2 = _make_agg(d2, n_edges, nb_max)(g2, src, dst, zeros2)
    out = _tc_final(agg2, h2, dinv, b2, fc_W, fc_b)
    return out.reshape(1)


# speedup vs baseline: 42.8025x; 1.2620x over previous
import functools

import jax
import jax.numpy as jnp
from jax import lax
from jax.experimental import pallas as pl
from jax.experimental.pallas import tpu as pltpu
from jax.experimental.pallas import tpu_sc as plsc

N_NODES = 10000
DEG_W = 16
N_PAD = 10240
NC, NS = 2, 16
NW = NC * NS
EB = 128
SBK = 4
RPT = N_PAD // NS


def _sc_mesh():
    return plsc.VectorSubcoreMesh(
        core_axis_name="c", subcore_axis_name="s",
        num_cores=NC, num_subcores=NS)


def _num_batches(w, n_edges, nb_max, batch):
    per_tile = nb_max * batch
    real = jnp.clip(n_edges - w * per_tile, 0, per_tile)
    return (real + batch - 1) // batch


def _make_deg(n_edges, nb_max):
    @functools.partial(
        pl.kernel,
        out_type=jax.ShapeDtypeStruct((NC, N_PAD, DEG_W), jnp.float32),
        mesh=_sc_mesh(),
        compiler_params=pltpu.CompilerParams(use_tc_tiling_on_sc=False),
        scratch_types=[
            pltpu.VMEM((nb_max, EB), jnp.int32),
            pltpu.VMEM((EB, DEG_W), jnp.float32),
            pltpu.VMEM_SHARED((N_PAD, DEG_W), jnp.float32),
        ],
    )
    def deg_k(dst_hbm, ones_hbm, zeros_hbm, out_hbm, dst_v, ones_v, acc_sh):
        cid = lax.axis_index("c")
        sid = lax.axis_index("s")
        w = cid * NS + sid
        pltpu.sync_copy(dst_hbm.at[w], dst_v)
        pltpu.sync_copy(ones_hbm, ones_v)
        pltpu.sync_copy(zeros_hbm, acc_sh.at[pl.ds(sid * RPT, RPT)])
        plsc.subcore_barrier()
        nb = _num_batches(w, n_edges, nb_max, EB)

        def body(j, carry):
            pltpu.sync_copy(ones_v, acc_sh.at[dst_v.at[j]], add=True)
            return carry

        lax.fori_loop(0, nb, body, 0)
        plsc.subcore_barrier()
        pltpu.sync_copy(acc_sh.at[pl.ds(sid * RPT, RPT)],
                        out_hbm.at[cid, pl.ds(sid * RPT, RPT)])

    return deg_k


def _make_agg(d_feat, n_edges, nb_max):
    @functools.partial(
        pl.kernel,
        out_type=jax.ShapeDtypeStruct((NC, N_PAD, d_feat), jnp.float32),
        mesh=_sc_mesh(),
        compiler_params=pltpu.CompilerParams(use_tc_tiling_on_sc=False),
        scratch_types=[
            pltpu.VMEM((nb_max, SBK * EB), jnp.int32),
            pltpu.VMEM((nb_max, SBK * EB), jnp.int32),
            pltpu.VMEM((SBK * EB, d_feat), jnp.float32),
            pltpu.VMEM((SBK * EB, d_feat), jnp.float32),
            pltpu.VMEM_SHARED((N_PAD, d_feat), jnp.float32),
            pltpu.SemaphoreType.DMA,
            pltpu.SemaphoreType.DMA,
        ],
    )
    def agg_k(g_hbm, src_hbm, dst_hbm, zeros_hbm, out_hbm,
              src_v, dst_v, rows0, rows1, acc_sh, sem0, sem1):
        cid = lax.axis_index("c")
        sid = lax.axis_index("s")
        w = cid * NS + sid
        pltpu.sync_copy(src_hbm.at[w], src_v)
        pltpu.sync_copy(dst_hbm.at[w], dst_v)
        pltpu.sync_copy(zeros_hbm, acc_sh.at[pl.ds(sid * RPT, RPT)])
        plsc.subcore_barrier()
        nb = _num_batches(w, n_edges, nb_max, SBK * EB)

        @pl.when(nb > 0)
        def _():
            pltpu.async_copy(g_hbm.at[src_v.at[0]], rows0, sem0)

        def stage(j, rows_cur, sem_cur, rows_nxt, sem_nxt):
            pltpu.make_async_copy(g_hbm.at[src_v.at[j]], rows_cur,
                                  sem_cur).wait()

            @pl.when(j + 1 < nb)
            def _():
                pltpu.async_copy(g_hbm.at[src_v.at[j + 1]], rows_nxt, sem_nxt)

            pltpu.sync_copy(rows_cur, acc_sh.at[dst_v.at[j]], add=True)

        def body(j, carry):
            @pl.when(j % 2 == 0)
            def _():
                stage(j, rows0, sem0, rows1, sem1)

            @pl.when(j % 2 == 1)
            def _():
                stage(j, rows1, sem1, rows0, sem0)

            return carry

        lax.fori_loop(0, nb, body, 0)
        plsc.subcore_barrier()
        pltpu.sync_copy(acc_sh.at[pl.ds(sid * RPT, RPT)],
                        out_hbm.at[cid, pl.ds(sid * RPT, RPT)])

    return agg_k


def _tc_prep1(deg2, x, w1):
    n, d_in = x.shape
    d_out = w1.shape[1]

    def body(deg_ref, x_ref, w_ref, dinv_ref, h_ref, g_ref):
        nr = x_ref.shape[0]
        deg = deg_ref[0, :nr, 0:1] + deg_ref[1, :nr, 0:1] + 1.0
        dinv = lax.rsqrt(deg)
        h = jnp.dot(x_ref[...], w_ref[...], preferred_element_type=jnp.float32,
                    precision=lax.Precision.HIGHEST)
        dinv_ref[...] = dinv
        h_ref[...] = h
        g_ref[...] = h * dinv

    return pl.pallas_call(
        body,
        out_shape=(
            jax.ShapeDtypeStruct((n, 1), jnp.float32),
            jax.ShapeDtypeStruct((n, d_out), jnp.float32),
            jax.ShapeDtypeStruct((n, d_out), jnp.float32),
        ),
    )(deg2, x, w1)


def _tc_mid(agg2, h1, dinv, b1, w2):
    n, d1 = h1.shape
    d2 = w2.shape[1]

    def body(agg_ref, h1_ref, dinv_ref, b1_ref, w2_ref, h2_ref, g2_ref):
        dinv_v = dinv_ref[...]
        n_rows = h1_ref.shape[0]
        pre = (dinv_v * (agg_ref[0, :n_rows] + agg_ref[1, :n_rows])
               + (dinv_v * dinv_v) * h1_ref[...] + b1_ref[...])
        out1 = jnp.maximum(pre, 0.0)
        h2 = jnp.dot(out1, w2_ref[...], preferred_element_type=jnp.float32,
                    precision=lax.Precision.HIGHEST)
        h2_ref[...] = h2
        g2_ref[...] = h2 * dinv_v

    return pl.pallas_call(
        body,
        out_shape=(
            jax.ShapeDtypeStruct((n, d2), jnp.float32),
            jax.ShapeDtypeStruct((n, d2), jnp.float32),
        ),
    )(agg2, h1, dinv, b1, w2)


def _tc_final(agg2, h2, dinv, b2, fc_w, fc_b):
    n, d2 = h2.shape

    def body(agg_ref, h2_ref, dinv_ref, b2_ref, fcw_ref, fcb_ref, out_ref):
        dinv_v = dinv_ref[...]
        n_rows = h2_ref.shape[0]
        pre = (dinv_v * (agg_ref[0, :n_rows] + agg_ref[1, :n_rows])
               + (dinv_v * dinv_v) * h2_ref[...] + b2_ref[...])
        out2 = jnp.maximum(pre, 0.0)
        pooled = jnp.sum(out2, axis=0, keepdims=True) * (1.0 / n)
        val = jnp.dot(pooled, fcw_ref[...], preferred_element_type=jnp.float32,
                    precision=lax.Precision.HIGHEST)
        out_ref[...] = val + fcb_ref[...]

    return pl.pallas_call(
        body,
        out_shape=jax.ShapeDtypeStruct((1, 1), jnp.float32),
    )(agg2, h2, dinv, b2, fc_w, fc_b)


def kernel(x, edge_index, W1, b1, W2, b2, fc_W, fc_b):
    n_edges = edge_index.shape[1]
    chunk = NW * SBK * EB
    e_pad = ((n_edges + chunk - 1) // chunk) * chunk
    nb_max = e_pad // chunk
    pad = e_pad - n_edges
    src = jnp.concatenate(
        [edge_index[0], jnp.zeros((pad,), jnp.int32)]
    ).reshape(NW, nb_max, SBK * EB)
    dst = jnp.concatenate(
        [edge_index[1], jnp.zeros((pad,), jnp.int32)]
    ).reshape(NW, nb_max, SBK * EB)

    ones_col = jnp.ones((EB, DEG_W), jnp.float32)
    zeros_col = jnp.zeros((RPT, DEG_W), jnp.float32)

    dst_flat = dst.reshape(NW, nb_max * SBK, EB)
    deg2 = _make_deg(n_edges, nb_max * SBK)(dst_flat, ones_col, zeros_col)
    dinv, h1, g1 = _tc_prep1(deg2, x, W1)

    d1 = W1.shape[1]
    zeros1 = jnp.zeros((RPT, d1), jnp.float32)
    agg1 = _make_agg(d1, n_edges, nb_max)(g1, src, dst, zeros1)
    h2, g2 = _tc_mid(agg1, h1, dinv, b1, W2)

    d2 = W2.shape[1]
    zeros2 = jnp.zeros((RPT, d2), jnp.float32)
    agg2 = _make_agg(d2, n_edges, nb_max)(g2, src, dst, zeros2)
    out = _tc_final(agg2, h2, dinv, b2, fc_W, fc_b)
    return out.reshape(1)


# speedup vs baseline: 46.2536x; 1.0806x over previous
import functools

import jax
import jax.numpy as jnp
from jax import lax
from jax.experimental import pallas as pl
from jax.experimental.pallas import tpu as pltpu
from jax.experimental.pallas import tpu_sc as plsc

N_NODES = 10000
DEG_W = 16
N_PAD = 10240
NC, NS = 2, 16
NW = NC * NS
EB = 512
RPT = N_PAD // NS
BR = 1280
GR = N_PAD // BR
PREC = lax.Precision.HIGHEST


def _sc_mesh():
    return plsc.VectorSubcoreMesh(
        core_axis_name="c", subcore_axis_name="s",
        num_cores=NC, num_subcores=NS)


def _worker_span(w, nb_tot, cp):
    q, r = divmod(nb_tot, NW)
    nb = q + jnp.where(w < r, 1, 0)
    off = w * q + jnp.minimum(w, r)
    copyoff = jnp.minimum(off, nb_tot - cp)
    return nb, off - copyoff, copyoff


def _make_deg(nb_tot, cp):
    @functools.partial(
        pl.kernel,
        out_type=jax.ShapeDtypeStruct((NC, N_PAD, DEG_W), jnp.float32),
        mesh=_sc_mesh(),
        compiler_params=pltpu.CompilerParams(use_tc_tiling_on_sc=False),
        scratch_types=[
            pltpu.VMEM((cp, EB), jnp.int32),
            pltpu.VMEM((EB, DEG_W), jnp.float32),
            pltpu.VMEM_SHARED((N_PAD, DEG_W), jnp.float32),
        ],
    )
    def deg_k(ei_hbm, ones_hbm, zeros_hbm, out_hbm, dst_v, ones_v, acc_sh):
        cid = lax.axis_index("c")
        sid = lax.axis_index("s")
        w = cid * NS + sid
        nb, base, copyoff = _worker_span(w, nb_tot, cp)
        pltpu.sync_copy(ei_hbm.at[1, pl.ds(copyoff, cp)], dst_v)
        pltpu.sync_copy(ones_hbm, ones_v)
        pltpu.sync_copy(zeros_hbm, acc_sh.at[pl.ds(sid * RPT, RPT)])
        plsc.subcore_barrier()

        def body(j, carry):
            pltpu.sync_copy(ones_v, acc_sh.at[dst_v.at[base + j]], add=True)
            return carry

        lax.fori_loop(0, nb, body, 0)
        plsc.subcore_barrier()
        pltpu.sync_copy(acc_sh.at[pl.ds(sid * RPT, RPT)],
                        out_hbm.at[cid, pl.ds(sid * RPT, RPT)])

    return deg_k


def _make_agg(d_feat, nb_tot, cp):
    @functools.partial(
        pl.kernel,
        out_type=jax.ShapeDtypeStruct((NC, N_PAD, d_feat), jnp.float32),
        mesh=_sc_mesh(),
        compiler_params=pltpu.CompilerParams(use_tc_tiling_on_sc=False),
        scratch_types=[
            pltpu.VMEM((cp, EB), jnp.int32),
            pltpu.VMEM((cp, EB), jnp.int32),
            pltpu.VMEM((EB, d_feat), jnp.float32),
            pltpu.VMEM((EB, d_feat), jnp.float32),
            pltpu.VMEM_SHARED((N_PAD, d_feat), jnp.float32),
            pltpu.SemaphoreType.DMA,
            pltpu.SemaphoreType.DMA,
        ],
    )
    def agg_k(g_hbm, ei_hbm, zeros_hbm, out_hbm,
              src_v, dst_v, rows0, rows1, acc_sh, sem0, sem1):
        cid = lax.axis_index("c")
        sid = lax.axis_index("s")
        w = cid * NS + sid
        nb, base, copyoff = _worker_span(w, nb_tot, cp)
        pltpu.sync_copy(ei_hbm.at[0, pl.ds(copyoff, cp)], src_v)
        pltpu.sync_copy(ei_hbm.at[1, pl.ds(copyoff, cp)], dst_v)
        pltpu.sync_copy(zeros_hbm, acc_sh.at[pl.ds(sid * RPT, RPT)])
        plsc.subcore_barrier()

        @pl.when(nb > 0)
        def _():
            pltpu.async_copy(g_hbm.at[src_v.at[base]], rows0, sem0)

        def stage(j, rows_cur, sem_cur, rows_nxt, sem_nxt):
            pltpu.make_async_copy(g_hbm.at[src_v.at[base + j]], rows_cur,
                                  sem_cur).wait()

            @pl.when(j + 1 < nb)
            def _():
                pltpu.async_copy(g_hbm.at[src_v.at[base + j + 1]],
                                 rows_nxt, sem_nxt)

            pltpu.sync_copy(rows_cur, acc_sh.at[dst_v.at[base + j]], add=True)

        def body(j, carry):
            @pl.when(j % 2 == 0)
            def _():
                stage(j, rows0, sem0, rows1, sem1)

            @pl.when(j % 2 == 1)
            def _():
                stage(j, rows1, sem1, rows0, sem0)

            return carry

        lax.fori_loop(0, nb, body, 0)
        plsc.subcore_barrier()
        pltpu.sync_copy(acc_sh.at[pl.ds(sid * RPT, RPT)],
                        out_hbm.at[cid, pl.ds(sid * RPT, RPT)])

    return agg_k


def _tc_matmul(x, w1):
    n, d_in = x.shape
    d_out = w1.shape[1]

    def body(x_ref, w_ref, h_ref):
        h_ref[...] = jnp.dot(x_ref[...], w_ref[...],
                             preferred_element_type=jnp.float32,
                             precision=PREC)

    return pl.pallas_call(
        body,
        grid=(GR,),
        in_specs=[
            pl.BlockSpec((BR, d_in), lambda i: (i, 0)),
            pl.BlockSpec((d_in, d_out), lambda i: (0, 0)),
        ],
        out_specs=pl.BlockSpec((BR, d_out), lambda i: (i, 0)),
        out_shape=jax.ShapeDtypeStruct((n, d_out), jnp.float32),
    )(x, w1)


def _tc_scale(deg2, h):
    n, d_out = h.shape

    def body(deg_ref, h_ref, dinv_ref, g_ref):
        deg = deg_ref[0, :, 0:1] + deg_ref[1, :, 0:1] + 1.0
        dinv = lax.rsqrt(deg)
        dinv_ref[...] = dinv
        g_ref[...] = h_ref[...] * dinv

    return pl.pallas_call(
        body,
        grid=(GR,),
        in_specs=[
            pl.BlockSpec((NC, BR, DEG_W), lambda i: (0, i, 0)),
            pl.BlockSpec((BR, d_out), lambda i: (i, 0)),
        ],
        out_specs=(
            pl.BlockSpec((BR, 1), lambda i: (i, 0)),
            pl.BlockSpec((BR, d_out), lambda i: (i, 0)),
        ),
        out_shape=(
            jax.ShapeDtypeStruct((n, 1), jnp.float32),
            jax.ShapeDtypeStruct((n, d_out), jnp.float32),
        ),
    )(deg2, h)


def _tc_mid(agg2, h1, dinv, b1, w2):
    n, d1 = h1.shape
    d2 = w2.shape[1]

    def body(agg_ref, h1_ref, dinv_ref, b1_ref, w2_ref, h2_ref, g2_ref):
        dinv_v = dinv_ref[...]
        pre = (dinv_v * (agg_ref[0] + agg_ref[1])
               + (dinv_v * dinv_v) * h1_ref[...] + b1_ref[...])
        out1 = jnp.maximum(pre, 0.0)
        h2 = jnp.dot(out1, w2_ref[...], preferred_element_type=jnp.float32,
                     precision=PREC)
        h2_ref[...] = h2
        g2_ref[...] = h2 * dinv_v

    return pl.pallas_call(
        body,
        grid=(GR,),
        in_specs=[
            pl.BlockSpec((NC, BR, d1), lambda i: (0, i, 0)),
            pl.BlockSpec((BR, d1), lambda i: (i, 0)),
            pl.BlockSpec((BR, 1), lambda i: (i, 0)),
            pl.BlockSpec((1, d1), lambda i: (0, 0)),
            pl.BlockSpec((d1, d2), lambda i: (0, 0)),
        ],
        out_specs=(
            pl.BlockSpec((BR, d2), lambda i: (i, 0)),
            pl.BlockSpec((BR, d2), lambda i: (i, 0)),
        ),
        out_shape=(
            jax.ShapeDtypeStruct((n, d2), jnp.float32),
            jax.ShapeDtypeStruct((n, d2), jnp.float32),
        ),
    )(agg2, h1, dinv, b1, w2)


def _tc_final(agg2, h2, dinv, b2, fc_w, fc_b):
    n, d2 = h2.shape

    def body(agg_ref, h2_ref, dinv_ref, b2_ref, fcw_ref, fcb_ref, out_ref,
             pool_ref):
        i = pl.program_id(0)
        dinv_v = dinv_ref[...]
        pre = (dinv_v * (agg_ref[0] + agg_ref[1])
               + (dinv_v * dinv_v) * h2_ref[...] + b2_ref[...])
        out2 = jnp.maximum(pre, 0.0)
        row = i * BR + lax.broadcasted_iota(jnp.int32, (BR, d2), 0)
        out2 = jnp.where(row < n, out2, 0.0)
        part = jnp.sum(out2, axis=0, keepdims=True)

        @pl.when(i == 0)
        def _():
            pool_ref[...] = part

        @pl.when(i > 0)
        def _():
            pool_ref[...] = pool_ref[...] + part

        @pl.when(i == GR - 1)
        def _():
            pooled = pool_ref[...] * (1.0 / n)
            val = jnp.dot(pooled, fcw_ref[...],
                          preferred_element_type=jnp.float32, precision=PREC)
            out_ref[...] = val + fcb_ref[...]

    return pl.pallas_call(
        body,
        grid=(GR,),
        in_specs=[
            pl.BlockSpec((NC, BR, d2), lambda i: (0, i, 0)),
            pl.BlockSpec((BR, d2), lambda i: (i, 0)),
            pl.BlockSpec((BR, 1), lambda i: (i, 0)),
            pl.BlockSpec((1, d2), lambda i: (0, 0)),
            pl.BlockSpec((d2, 1), lambda i: (0, 0)),
            pl.BlockSpec((1, 1), lambda i: (0, 0)),
        ],
        out_specs=pl.BlockSpec((1, 1), lambda i: (0, 0)),
        out_shape=jax.ShapeDtypeStruct((1, 1), jnp.float32),
        scratch_shapes=[pltpu.VMEM((1, d2), jnp.float32)],
    )(agg2, h2, dinv, b2, fc_w, fc_b)


def kernel(x, edge_index, W1, b1, W2, b2, fc_W, fc_b):
    n_edges = edge_index.shape[1]
    e_pad = ((n_edges + EB - 1) // EB) * EB
    if e_pad != n_edges:
        fill = jnp.array([[0], [N_PAD - 1]], jnp.int32)
        pad = jnp.broadcast_to(fill, (2, e_pad - n_edges))
        edge_index = jnp.concatenate([edge_index, pad], axis=1)
    nb_tot = e_pad // EB
    cp = nb_tot // NW + (1 if nb_tot % NW else 0)
    ei = edge_index.reshape(2, nb_tot, EB)

    ones_row = jnp.ones((EB, DEG_W), jnp.float32)
    zeros_deg = jnp.zeros((RPT, DEG_W), jnp.float32)

    deg2 = _make_deg(nb_tot, cp)(ei, ones_row, zeros_deg)
    h1 = _tc_matmul(x, W1)
    dinv, g1 = _tc_scale(deg2, h1)

    d1 = W1.shape[1]
    zeros1 = jnp.zeros((RPT, d1), jnp.float32)
    agg1 = _make_agg(d1, nb_tot, cp)(g1, ei, zeros1)
    h2, g2 = _tc_mid(agg1, h1, dinv, b1.reshape(1, d1), W2)

    d2 = W2.shape[1]
    zeros2 = jnp.zeros((RPT, d2), jnp.float32)
    agg2 = _make_agg(d2, nb_tot, cp)(g2, ei, zeros2)
    out = _tc_final(agg2, h2, dinv, b2.reshape(1, d2), fc_W,
                    fc_b.reshape(1, 1))
    return out.reshape(1)


# speedup vs baseline: 47.1640x; 1.0197x over previous
import functools

import jax
import jax.numpy as jnp
from jax import lax
from jax.experimental import pallas as pl
from jax.experimental.pallas import tpu as pltpu
from jax.experimental.pallas import tpu_sc as plsc

N_NODES = 10000
DEG_W = 16
N_PAD = 10240
NC, NS = 2, 16
NW = NC * NS
EB = 512
RPT = N_PAD // NS
BR = 2560
GR = N_PAD // BR
PREC = lax.Precision.HIGHEST


def _sc_mesh():
    return plsc.VectorSubcoreMesh(
        core_axis_name="c", subcore_axis_name="s",
        num_cores=NC, num_subcores=NS)


def _worker_span(w, nb_tot, cp, align=1):
    q, r = divmod(nb_tot, NW)
    nb = q + jnp.where(w < r, 1, 0)
    off = w * q + jnp.minimum(w, r)
    if align == 1:
        copyoff = jnp.minimum(off, nb_tot - cp)
    else:
        pad8 = -(-nb_tot // align) * align
        copyoff = jnp.minimum(
            pl.multiple_of((off // align) * align, align), pad8 - cp)
    return nb, off - copyoff, copyoff


def _deg_window(nb_tot):
    q = nb_tot // NW
    return -(-(q + 9) // 8) * 8


def _make_deg(nb_tot, cp):
    @functools.partial(
        pl.kernel,
        out_type=jax.ShapeDtypeStruct((NC, N_PAD, DEG_W), jnp.float32),
        mesh=_sc_mesh(),
        compiler_params=pltpu.CompilerParams(use_tc_tiling_on_sc=False),
        scratch_types=[
            pltpu.VMEM((cp, EB), jnp.int32),
            pltpu.VMEM((EB, DEG_W), jnp.float32),
            pltpu.VMEM_SHARED((N_PAD, DEG_W), jnp.float32),
        ],
    )
    def deg_k(ei_hbm, ones_hbm, zeros_hbm, out_hbm, dst_v, ones_v, acc_sh):
        cid = lax.axis_index("c")
        sid = lax.axis_index("s")
        w = cid * NS + sid
        nb, base, copyoff = _worker_span(w, nb_tot, cp)
        pltpu.sync_copy(ei_hbm.at[1, pl.ds(copyoff, cp)], dst_v)
        pltpu.sync_copy(ones_hbm, ones_v)
        pltpu.sync_copy(zeros_hbm, acc_sh.at[pl.ds(sid * RPT, RPT)])
        plsc.subcore_barrier()

        def body(j, carry):
            pltpu.sync_copy(ones_v, acc_sh.at[dst_v.at[base + j]], add=True)
            return carry

        lax.fori_loop(0, nb, body, 0)
        plsc.subcore_barrier()
        pltpu.sync_copy(acc_sh.at[pl.ds(sid * RPT, RPT)],
                        out_hbm.at[cid, pl.ds(sid * RPT, RPT)])

    return deg_k


def _make_agg(d_feat, nb_tot, cp):
    @functools.partial(
        pl.kernel,
        out_type=jax.ShapeDtypeStruct((NC, N_PAD, d_feat), jnp.float32),
        mesh=_sc_mesh(),
        compiler_params=pltpu.CompilerParams(use_tc_tiling_on_sc=False),
        scratch_types=[
            pltpu.VMEM((cp, EB), jnp.int32),
            pltpu.VMEM((cp, EB), jnp.int32),
            pltpu.VMEM((EB, d_feat), jnp.float32),
            pltpu.VMEM((EB, d_feat), jnp.float32),
            pltpu.VMEM_SHARED((N_PAD, d_feat), jnp.float32),
            pltpu.SemaphoreType.DMA,
            pltpu.SemaphoreType.DMA,
        ],
    )
    def agg_k(g_hbm, ei_hbm, zeros_hbm, out_hbm,
              src_v, dst_v, rows0, rows1, acc_sh, sem0, sem1):
        cid = lax.axis_index("c")
        sid = lax.axis_index("s")
        w = cid * NS + sid
        nb, base, copyoff = _worker_span(w, nb_tot, cp)
        pltpu.sync_copy(ei_hbm.at[0, pl.ds(copyoff, cp)], src_v)
        pltpu.sync_copy(ei_hbm.at[1, pl.ds(copyoff, cp)], dst_v)
        pltpu.sync_copy(zeros_hbm, acc_sh.at[pl.ds(sid * RPT, RPT)])
        plsc.subcore_barrier()

        @pl.when(nb > 0)
        def _():
            pltpu.async_copy(g_hbm.at[src_v.at[base]], rows0, sem0)

        def stage(j, rows_cur, sem_cur, rows_nxt, sem_nxt):
            pltpu.make_async_copy(g_hbm.at[src_v.at[base + j]], rows_cur,
                                  sem_cur).wait()

            @pl.when(j + 1 < nb)
            def _():
                pltpu.async_copy(g_hbm.at[src_v.at[base + j + 1]],
                                 rows_nxt, sem_nxt)

            pltpu.sync_copy(rows_cur, acc_sh.at[dst_v.at[base + j]], add=True)

        def body(j, carry):
            @pl.when(j % 2 == 0)
            def _():
                stage(j, rows0, sem0, rows1, sem1)

            @pl.when(j % 2 == 1)
            def _():
                stage(j, rows1, sem1, rows0, sem0)

            return carry

        lax.fori_loop(0, nb, body, 0)
        plsc.subcore_barrier()
        pltpu.sync_copy(acc_sh.at[pl.ds(sid * RPT, RPT)],
                        out_hbm.at[cid, pl.ds(sid * RPT, RPT)])

    return agg_k


def _tc_matmul(x, w1):
    n, d_in = x.shape
    d_out = w1.shape[1]

    def body(x_ref, w_ref, h_ref):
        h_ref[...] = jnp.dot(x_ref[...], w_ref[...],
                             preferred_element_type=jnp.float32,
                             precision=PREC)

    return pl.pallas_call(
        body,
        grid=(GR,),
        in_specs=[
            pl.BlockSpec((BR, d_in), lambda i: (i, 0)),
            pl.BlockSpec((d_in, d_out), lambda i: (0, 0)),
        ],
        out_specs=pl.BlockSpec((BR, d_out), lambda i: (i, 0)),
        out_shape=jax.ShapeDtypeStruct((n, d_out), jnp.float32),
    )(x, w1)


def _tc_scale(deg2, h):
    n, d_out = h.shape

    def body(deg_ref, h_ref, dinv_ref, g_ref):
        deg = deg_ref[0, :, 0:1] + deg_ref[1, :, 0:1] + 1.0
        dinv = lax.rsqrt(deg)
        dinv_ref[...] = dinv
        g_ref[...] = h_ref[...] * dinv

    return pl.pallas_call(
        body,
        grid=(GR,),
        in_specs=[
            pl.BlockSpec((NC, BR, DEG_W), lambda i: (0, i, 0)),
            pl.BlockSpec((BR, d_out), lambda i: (i, 0)),
        ],
        out_specs=(
            pl.BlockSpec((BR, 1), lambda i: (i, 0)),
            pl.BlockSpec((BR, d_out), lambda i: (i, 0)),
        ),
        out_shape=(
            jax.ShapeDtypeStruct((n, 1), jnp.float32),
            jax.ShapeDtypeStruct((n, d_out), jnp.float32),
        ),
    )(deg2, h)


def _tc_mid(agg2, h1, dinv, b1, w2):
    n, d1 = h1.shape
    d2 = w2.shape[1]

    def body(agg_ref, h1_ref, dinv_ref, b1_ref, w2_ref, h2_ref, g2_ref):
        dinv_v = dinv_ref[...]
        pre = (dinv_v * (agg_ref[0] + agg_ref[1])
               + (dinv_v * dinv_v) * h1_ref[...] + b1_ref[...])
        out1 = jnp.maximum(pre, 0.0)
        h2 = jnp.dot(out1, w2_ref[...], preferred_element_type=jnp.float32,
                     precision=PREC)
        h2_ref[...] = h2
        g2_ref[...] = h2 * dinv_v

    return pl.pallas_call(
        body,
        grid=(GR,),
        in_specs=[
            pl.BlockSpec((NC, BR, d1), lambda i: (0, i, 0)),
            pl.BlockSpec((BR, d1), lambda i: (i, 0)),
            pl.BlockSpec((BR, 1), lambda i: (i, 0)),
            pl.BlockSpec((1, d1), lambda i: (0, 0)),
            pl.BlockSpec((d1, d2), lambda i: (0, 0)),
        ],
        out_specs=(
            pl.BlockSpec((BR, d2), lambda i: (i, 0)),
            pl.BlockSpec((BR, d2), lambda i: (i, 0)),
        ),
        out_shape=(
            jax.ShapeDtypeStruct((n, d2), jnp.float32),
            jax.ShapeDtypeStruct((n, d2), jnp.float32),
        ),
    )(agg2, h1, dinv, b1, w2)


def _tc_final(agg2, h2, dinv, b2, fc_w, fc_b):
    n, d2 = h2.shape

    def body(agg_ref, h2_ref, dinv_ref, b2_ref, fcw_ref, fcb_ref, out_ref,
             pool_ref):
        i = pl.program_id(0)
        dinv_v = dinv_ref[...]
        pre = (dinv_v * (agg_ref[0] + agg_ref[1])
               + (dinv_v * dinv_v) * h2_ref[...] + b2_ref[...])
        out2 = jnp.maximum(pre, 0.0)
        row = i * BR + lax.broadcasted_iota(jnp.int32, (BR, d2), 0)
        out2 = jnp.where(row < n, out2, 0.0)
        part = jnp.sum(out2, axis=0, keepdims=True)

        @pl.when(i == 0)
        def _():
            pool_ref[...] = part

        @pl.when(i > 0)
        def _():
            pool_ref[...] = pool_ref[...] + part

        @pl.when(i == GR - 1)
        def _():
            pooled = pool_ref[...] * (1.0 / n)
            val = jnp.dot(pooled, fcw_ref[...],
                          preferred_element_type=jnp.float32, precision=PREC)
            out_ref[...] = val + fcb_ref[...]

    return pl.pallas_call(
        body,
        grid=(GR,),
        in_specs=[
            pl.BlockSpec((NC, BR, d2), lambda i: (0, i, 0)),
            pl.BlockSpec((BR, d2), lambda i: (i, 0)),
            pl.BlockSpec((BR, 1), lambda i: (i, 0)),
            pl.BlockSpec((1, d2), lambda i: (0, 0)),
            pl.BlockSpec((d2, 1), lambda i: (0, 0)),
            pl.BlockSpec((1, 1), lambda i: (0, 0)),
        ],
        out_specs=pl.BlockSpec((1, 1), lambda i: (0, 0)),
        out_shape=jax.ShapeDtypeStruct((1, 1), jnp.float32),
        scratch_shapes=[pltpu.VMEM((1, d2), jnp.float32)],
    )(agg2, h2, dinv, b2, fc_w, fc_b)


def kernel(x, edge_index, W1, b1, W2, b2, fc_W, fc_b):
    n_edges = edge_index.shape[1]
    e_pad = ((n_edges + EB - 1) // EB) * EB
    if e_pad != n_edges:
        fill = jnp.array([[0], [N_PAD - 1]], jnp.int32)
        pad = jnp.broadcast_to(fill, (2, e_pad - n_edges))
        edge_index = jnp.concatenate([edge_index, pad], axis=1)
    nb_tot = e_pad // EB
    cp = nb_tot // NW + (1 if nb_tot % NW else 0)
    ei = edge_index.reshape(2, nb_tot, EB)

    ones_row = jnp.ones((EB, DEG_W), jnp.float32)
    zeros_deg = jnp.zeros((RPT, DEG_W), jnp.float32)

    deg2 = _make_deg(nb_tot, cp)(ei, ones_row, zeros_deg)
    h1 = _tc_matmul(x, W1)
    dinv, g1 = _tc_scale(deg2, h1)

    d1 = W1.shape[1]
    zeros1 = jnp.zeros((RPT, d1), jnp.float32)
    agg1 = _make_agg(d1, nb_tot, cp)(g1, ei, zeros1)
    h2, g2 = _tc_mid(agg1, h1, dinv, b1.reshape(1, d1), W2)

    d2 = W2.shape[1]
    zeros2 = jnp.zeros((RPT, d2), jnp.float32)
    agg2 = _make_agg(d2, nb_tot, cp)(g2, ei, zeros2)
    out = _tc_final(agg2, h2, dinv, b2.reshape(1, d2), fc_W,
                    fc_b.reshape(1, 1))
    return out.reshape(1)


# speedup vs baseline: 48.8413x; 1.0356x over previous
import functools

import jax
import jax.numpy as jnp
from jax import lax
from jax.experimental import pallas as pl
from jax.experimental.pallas import tpu as pltpu
from jax.experimental.pallas import tpu_sc as plsc

N_NODES = 10000
DEG_W = 16
N_PAD = 10240
NC, NS = 2, 16
NW = NC * NS
EB = 512
RPT = N_PAD // NS
BR = 2560
GR = N_PAD // BR
PREC = lax.Precision.HIGHEST


def _sc_mesh():
    return plsc.VectorSubcoreMesh(
        core_axis_name="c", subcore_axis_name="s",
        num_cores=NC, num_subcores=NS)


def _worker_span(w, nb_tot, cp, align=1):
    q, r = divmod(nb_tot, NW)
    nb = q + jnp.where(w < r, 1, 0)
    off = w * q + jnp.minimum(w, r)
    if align == 1:
        copyoff = jnp.minimum(off, nb_tot - cp)
    else:
        pad8 = -(-nb_tot // align) * align
        copyoff = jnp.minimum(
            pl.multiple_of((off // align) * align, align), pad8 - cp)
    return nb, off - copyoff, copyoff


def _deg_window(nb_tot):
    q = nb_tot // NW
    return -(-(q + 9) // 8) * 8


def _make_deg(nb_tot, cp):
    @functools.partial(
        pl.kernel,
        out_type=jax.ShapeDtypeStruct((NC, N_PAD, DEG_W), jnp.float32),
        mesh=_sc_mesh(),
        compiler_params=pltpu.CompilerParams(use_tc_tiling_on_sc=False),
        scratch_types=[
            pltpu.VMEM((cp, EB), jnp.int32),
            pltpu.VMEM((EB, DEG_W), jnp.float32),
            pltpu.VMEM_SHARED((N_PAD, DEG_W), jnp.float32),
        ],
    )
    def deg_k(ei_hbm, ones_hbm, zeros_hbm, out_hbm, dst_v, ones_v, acc_sh):
        cid = lax.axis_index("c")
        sid = lax.axis_index("s")
        w = cid * NS + sid
        nb, base, copyoff = _worker_span(w, nb_tot, cp)
        pltpu.sync_copy(ei_hbm.at[1, pl.ds(copyoff, cp)], dst_v)
        pltpu.sync_copy(ones_hbm, ones_v)
        pltpu.sync_copy(zeros_hbm, acc_sh.at[pl.ds(sid * RPT, RPT)])
        plsc.subcore_barrier()

        def body(j, carry):
            pltpu.sync_copy(ones_v, acc_sh.at[dst_v.at[base + j]], add=True)
            return carry

        lax.fori_loop(0, nb, body, 0)
        plsc.subcore_barrier()
        pltpu.sync_copy(acc_sh.at[pl.ds(sid * RPT, RPT)],
                        out_hbm.at[cid, pl.ds(sid * RPT, RPT)])

    return deg_k


def _make_agg(d_feat, nb_tot, cp, nbuf):
    @functools.partial(
        pl.kernel,
        out_type=jax.ShapeDtypeStruct((NC, N_PAD, d_feat), jnp.float32),
        mesh=_sc_mesh(),
        compiler_params=pltpu.CompilerParams(use_tc_tiling_on_sc=False),
        scratch_types=[
            pltpu.VMEM((cp, EB), jnp.int32),
            pltpu.VMEM((cp, EB), jnp.int32),
        ] + [pltpu.VMEM((EB, d_feat), jnp.float32)] * nbuf + [
            pltpu.VMEM_SHARED((N_PAD, d_feat), jnp.float32),
        ] + [pltpu.SemaphoreType.DMA] * (2 * nbuf),
    )
    def agg_k(g_hbm, ei_hbm, zeros_hbm, out_hbm, src_v, dst_v, *scr):
        bufs = scr[:nbuf]
        acc_sh = scr[nbuf]
        gsems = scr[nbuf + 1:2 * nbuf + 1]
        ssems = scr[2 * nbuf + 1:]
        cid = lax.axis_index("c")
        sid = lax.axis_index("s")
        w = cid * NS + sid
        nb, base, copyoff = _worker_span(w, nb_tot, cp)
        pltpu.sync_copy(ei_hbm.at[0, pl.ds(copyoff, cp)], src_v)
        pltpu.sync_copy(ei_hbm.at[1, pl.ds(copyoff, cp)], dst_v)
        pltpu.sync_copy(zeros_hbm, acc_sh.at[pl.ds(sid * RPT, RPT)])
        plsc.subcore_barrier()

        for i in range(nbuf - 1):
            @pl.when(nb > i)
            def _(i=i):
                pltpu.async_copy(g_hbm.at[src_v.at[base + i]],
                                 bufs[i], gsems[i])

        def stage(j, k):
            km1 = (k - 1) % nbuf
            pltpu.make_async_copy(g_hbm.at[src_v.at[base + j]], bufs[k],
                                  gsems[k]).wait()

            @pl.when(j >= 1)
            def _():
                pltpu.make_async_copy(
                    bufs[km1], acc_sh.at[dst_v.at[base + j - 1]],
                    ssems[km1]).wait()

            pltpu.async_copy(bufs[k], acc_sh.at[dst_v.at[base + j]],
                             ssems[k], add=True)

            @pl.when(j + nbuf - 1 < nb)
            def _():
                pltpu.async_copy(g_hbm.at[src_v.at[base + j + nbuf - 1]],
                                 bufs[km1], gsems[km1])

        def body(j, carry):
            for k in range(nbuf):
                @pl.when(j % nbuf == k)
                def _(k=k):
                    stage(j, k)

            return carry

        lax.fori_loop(0, nb, body, 0)

        @pl.when(nb > 0)
        def _():
            last = nb - 1
            for kk in range(nbuf):
                @pl.when(last % nbuf == kk)
                def _(kk=kk):
                    pltpu.make_async_copy(
                        bufs[kk], acc_sh.at[dst_v.at[base + last]],
                        ssems[kk]).wait()

        plsc.subcore_barrier()
        pltpu.sync_copy(acc_sh.at[pl.ds(sid * RPT, RPT)],
                        out_hbm.at[cid, pl.ds(sid * RPT, RPT)])

    return agg_k


def _tc_matmul(x, w1):
    n, d_in = x.shape
    d_out = w1.shape[1]

    def body(x_ref, w_ref, h_ref):
        h_ref[...] = jnp.dot(x_ref[...], w_ref[...],
                             preferred_element_type=jnp.float32,
                             precision=PREC)

    return pl.pallas_call(
        body,
        grid=(GR,),
        in_specs=[
            pl.BlockSpec((BR, d_in), lambda i: (i, 0)),
            pl.BlockSpec((d_in, d_out), lambda i: (0, 0)),
        ],
        out_specs=pl.BlockSpec((BR, d_out), lambda i: (i, 0)),
        out_shape=jax.ShapeDtypeStruct((n, d_out), jnp.float32),
    )(x, w1)


def _tc_scale(deg2, h):
    n, d_out = h.shape

    def body(deg_ref, h_ref, dinv_ref, g_ref):
        deg = deg_ref[0, :, 0:1] + deg_ref[1, :, 0:1] + 1.0
        dinv = lax.rsqrt(deg)
        dinv_ref[...] = dinv
        g_ref[...] = h_ref[...] * dinv

    return pl.pallas_call(
        body,
        grid=(GR,),
        in_specs=[
            pl.BlockSpec((NC, BR, DEG_W), lambda i: (0, i, 0)),
            pl.BlockSpec((BR, d_out), lambda i: (i, 0)),
        ],
        out_specs=(
            pl.BlockSpec((BR, 1), lambda i: (i, 0)),
            pl.BlockSpec((BR, d_out), lambda i: (i, 0)),
        ),
        out_shape=(
            jax.ShapeDtypeStruct((n, 1), jnp.float32),
            jax.ShapeDtypeStruct((n, d_out), jnp.float32),
        ),
    )(deg2, h)


def _tc_mid(agg2, h1, dinv, b1, w2):
    n, d1 = h1.shape
    d2 = w2.shape[1]

    def body(agg_ref, h1_ref, dinv_ref, b1_ref, w2_ref, h2_ref, g2_ref):
        dinv_v = dinv_ref[...]
        pre = (dinv_v * (agg_ref[0] + agg_ref[1])
               + (dinv_v * dinv_v) * h1_ref[...] + b1_ref[...])
        out1 = jnp.maximum(pre, 0.0)
        h2 = jnp.dot(out1, w2_ref[...], preferred_element_type=jnp.float32,
                     precision=PREC)
        h2_ref[...] = h2
        g2_ref[...] = h2 * dinv_v

    return pl.pallas_call(
        body,
        grid=(GR,),
        in_specs=[
            pl.BlockSpec((NC, BR, d1), lambda i: (0, i, 0)),
            pl.BlockSpec((BR, d1), lambda i: (i, 0)),
            pl.BlockSpec((BR, 1), lambda i: (i, 0)),
            pl.BlockSpec((1, d1), lambda i: (0, 0)),
            pl.BlockSpec((d1, d2), lambda i: (0, 0)),
        ],
        out_specs=(
            pl.BlockSpec((BR, d2), lambda i: (i, 0)),
            pl.BlockSpec((BR, d2), lambda i: (i, 0)),
        ),
        out_shape=(
            jax.ShapeDtypeStruct((n, d2), jnp.float32),
            jax.ShapeDtypeStruct((n, d2), jnp.float32),
        ),
    )(agg2, h1, dinv, b1, w2)


def _tc_final(agg2, h2, dinv, b2, fc_w, fc_b):
    n, d2 = h2.shape

    def body(agg_ref, h2_ref, dinv_ref, b2_ref, fcw_ref, fcb_ref, out_ref,
             pool_ref):
        i = pl.program_id(0)
        dinv_v = dinv_ref[...]
        pre = (dinv_v * (agg_ref[0] + agg_ref[1])
               + (dinv_v * dinv_v) * h2_ref[...] + b2_ref[...])
        out2 = jnp.maximum(pre, 0.0)
        row = i * BR + lax.broadcasted_iota(jnp.int32, (BR, d2), 0)
        out2 = jnp.where(row < n, out2, 0.0)
        part = jnp.sum(out2, axis=0, keepdims=True)

        @pl.when(i == 0)
        def _():
            pool_ref[...] = part

        @pl.when(i > 0)
        def _():
            pool_ref[...] = pool_ref[...] + part

        @pl.when(i == GR - 1)
        def _():
            pooled = pool_ref[...] * (1.0 / n)
            val = jnp.dot(pooled, fcw_ref[...],
                          preferred_element_type=jnp.float32, precision=PREC)
            out_ref[...] = val + fcb_ref[...]

    return pl.pallas_call(
        body,
        grid=(GR,),
        in_specs=[
            pl.BlockSpec((NC, BR, d2), lambda i: (0, i, 0)),
            pl.BlockSpec((BR, d2), lambda i: (i, 0)),
            pl.BlockSpec((BR, 1), lambda i: (i, 0)),
            pl.BlockSpec((1, d2), lambda i: (0, 0)),
            pl.BlockSpec((d2, 1), lambda i: (0, 0)),
            pl.BlockSpec((1, 1), lambda i: (0, 0)),
        ],
        out_specs=pl.BlockSpec((1, 1), lambda i: (0, 0)),
        out_shape=jax.ShapeDtypeStruct((1, 1), jnp.float32),
        scratch_shapes=[pltpu.VMEM((1, d2), jnp.float32)],
    )(agg2, h2, dinv, b2, fc_w, fc_b)


def kernel(x, edge_index, W1, b1, W2, b2, fc_W, fc_b):
    n_edges = edge_index.shape[1]
    e_pad = ((n_edges + EB - 1) // EB) * EB
    if e_pad != n_edges:
        fill = jnp.array([[0], [N_PAD - 1]], jnp.int32)
        pad = jnp.broadcast_to(fill, (2, e_pad - n_edges))
        edge_index = jnp.concatenate([edge_index, pad], axis=1)
    nb_tot = e_pad // EB
    cp = nb_tot // NW + (1 if nb_tot % NW else 0)
    ei = edge_index.reshape(2, nb_tot, EB)

    ones_row = jnp.ones((EB, DEG_W), jnp.float32)
    zeros_deg = jnp.zeros((RPT, DEG_W), jnp.float32)

    deg2 = _make_deg(nb_tot, cp)(ei, ones_row, zeros_deg)
    h1 = _tc_matmul(x, W1)
    dinv, g1 = _tc_scale(deg2, h1)

    d1 = W1.shape[1]
    zeros1 = jnp.zeros((RPT, d1), jnp.float32)
    agg1 = _make_agg(d1, nb_tot, cp, 2)(g1, ei, zeros1)
    h2, g2 = _tc_mid(agg1, h1, dinv, b1.reshape(1, d1), W2)

    d2 = W2.shape[1]
    zeros2 = jnp.zeros((RPT, d2), jnp.float32)
    agg2 = _make_agg(d2, nb_tot, cp, 3)(g2, ei, zeros2)
    out = _tc_final(agg2, h2, dinv, b2.reshape(1, d2), fc_W,
                    fc_b.reshape(1, 1))
    return out.reshape(1)
